# baseline (speedup 1.0000x reference)
import functools

import jax
import jax.numpy as jnp
from jax import lax
from jax.experimental import pallas as pl
from jax.experimental.pallas import tpu as pltpu
from jax.experimental.pallas import tpu_sc as plsc

NUM_LAYERS = 12
DUAL = 2
POOL = 256
LENGTH = 8
HEADS = 12
HDIM = 64
BATCH = 128
TOPK = 2

NGROUP_IN = NUM_LAYERS * DUAL * LENGTH * HEADS
NGROUP_OUT = NUM_LAYERS * DUAL * TOPK * LENGTH * HEADS
IN_SLAB = HDIM * POOL
OUT_SLAB = HDIM * BATCH
NTRIPLE = NUM_LAYERS * LENGTH * HEADS


def _sc_lane_gather(vin, g_addr):
    info = plsc.get_sparse_core_info()
    nc, ns, nl = info.num_cores, info.num_subcores, info.num_lanes
    nw = nc * ns
    tpw = NTRIPLE // nw

    mesh = plsc.VectorSubcoreMesh(core_axis_name="c", subcore_axis_name="s")

    @functools.partial(
        pl.kernel,
        mesh=mesh,
        out_type=jax.ShapeDtypeStruct((NGROUP_OUT * OUT_SLAB,), jnp.float32),
        compiler_params=pltpu.CompilerParams(needs_layout_passes=False),
        scratch_types=[
            pltpu.VMEM((TOPK * DUAL * BATCH,), jnp.int32),
            pltpu.VMEM((DUAL * IN_SLAB,), jnp.float32),
            pltpu.VMEM((DUAL * IN_SLAB,), jnp.float32),
            pltpu.VMEM((TOPK * DUAL * OUT_SLAB,), jnp.float32),
            pltpu.SemaphoreType.DMA,
            pltpu.SemaphoreType.DMA,
            pltpu.SemaphoreType.DMA,
        ],
    )
    def body(vin_hbm, g_hbm, out_hbm, g_v, buf_a, buf_b, buf_out,
             sem_a, sem_b, sem_out):
        wid = lax.axis_index("s") * nc + lax.axis_index("c")
        pltpu.sync_copy(g_hbm, g_v)
        bufs = (buf_a, buf_b)
        sems = (sem_a, sem_b)

        def start_in(t, buf, sem):
            l = t // (LENGTH * HEADS)
            rem = t % (LENGTH * HEADS)
            g0 = l * (DUAL * LENGTH * HEADS) + rem
            for dd in range(DUAL):
                pltpu.async_copy(
                    vin_hbm.at[pl.ds((g0 + dd * (LENGTH * HEADS)) * IN_SLAB,
                                     IN_SLAB)],
                    buf.at[pl.ds(dd * IN_SLAB, IN_SLAB)], sem)

        def wait_in(buf, sem):
            for dd in range(DUAL):
                pltpu.make_async_copy(
                    vin_hbm.at[pl.ds(0, IN_SLAB)],
                    buf.at[pl.ds(dd * IN_SLAB, IN_SLAB)], sem).wait()

        def wait_out():
            for c in range(TOPK * DUAL):
                pltpu.make_async_copy(
                    buf_out.at[pl.ds(c * OUT_SLAB, OUT_SLAB)],
                    out_hbm.at[pl.ds(0, OUT_SLAB)], sem_out).wait()

        start_in(wid * tpw, buf_a, sem_a)

        @pl.loop(0, tpw, step=2)
        def outer(j):
            for bsel in range(2):
                je = j + bsel
                t = wid * tpw + je
                l = t // (LENGTH * HEADS)
                rem = t % (LENGTH * HEADS)
                s = rem // HEADS
                h = rem % HEADS
                buf_in = bufs[bsel]

                @pl.when(je + 1 < tpw)
                def _():
                    start_in(t + 1, bufs[1 - bsel], sems[1 - bsel])

                @pl.when(je > 0)
                def _():
                    wait_out()
                wait_in(buf_in, sems[bsel])

                for c in range(TOPK * DUAL):
                    dprime = c // TOPK
                    k = c % TOPK
                    for i in range(BATCH // 16):
                        gv = g_v[pl.ds(c * BATCH + i * 16, 16)]

                        @plsc.parallel_loop(0, HDIM, unroll=8)
                        def blk(e):
                            base = (e // 8) * (8 * POOL) + (e % 8) * BATCH
                            val = plsc.load_gather(buf_in, [gv + base])
                            buf_out[pl.ds(c * OUT_SLAB + e * BATCH + i * 16,
                                          16)] = val
                    q = (l * DUAL + dprime) * (TOPK * LENGTH * HEADS) \
                        + (k * LENGTH + s) * HEADS + h
                    pltpu.async_copy(
                        buf_out.at[pl.ds(c * OUT_SLAB, OUT_SLAB)],
                        out_hbm.at[pl.ds(q * OUT_SLAB, OUT_SLAB)], sem_out)

        wait_out()

    return body(vin, g_addr)


def kernel(x_embed, prompt_idx, prompt):
    del x_embed
    idx32 = prompt_idx.astype(jnp.int32)

    vin = prompt.transpose(0, 1, 3, 4, 5, 2)
    vin = vin.reshape(NUM_LAYERS, DUAL, LENGTH, HEADS, 8, 8, 2, 128)
    vin = vin.transpose(0, 1, 2, 3, 4, 6, 5, 7)
    vin = vin.reshape(NGROUP_IN * IN_SLAB)

    bprime = jnp.arange(BATCH, dtype=jnp.int32)
    dprime = jnp.array([0, 0, 1, 1], dtype=jnp.int32)
    kk = jnp.array([0, 1, 0, 1], dtype=jnp.int32)
    m = 2 * bprime[None, :] + dprime[:, None]
    d = m // BATCH
    b = m % BATCH
    p = idx32[b, kk[:, None]]
    g_addr = (d * IN_SLAB + (p // 128) * (8 * 128) + p % 128).reshape(-1)

    vout = _sc_lane_gather(vin, g_addr)

    out = vout.reshape(NUM_LAYERS, DUAL, TOPK * LENGTH, HEADS, HDIM, BATCH)
    out = out.transpose(0, 5, 1, 2, 3, 4)
    return (prompt_idx, out)


# speedup vs baseline: 1.3016x; 1.3016x over previous
import functools

import jax
import jax.numpy as jnp
from jax import lax
from jax.experimental import pallas as pl
from jax.experimental.pallas import tpu as pltpu
from jax.experimental.pallas import tpu_sc as plsc

NUM_LAYERS = 12
DUAL = 2
POOL = 256
LENGTH = 8
HEADS = 12
HDIM = 64
BATCH = 128
TOPK = 2

NGROUP_IN = NUM_LAYERS * DUAL * LENGTH * HEADS
NGROUP_OUT = NUM_LAYERS * DUAL * TOPK * LENGTH * HEADS
IN_SLAB = HDIM * POOL
OUT_SLAB = HDIM * BATCH
NTRIPLE = NUM_LAYERS * LENGTH * HEADS


def _sc_lane_gather(vin, g_addr):
    info = plsc.get_sparse_core_info()
    nc, ns, nl = info.num_cores, info.num_subcores, info.num_lanes
    nw = nc * ns
    tpw = NTRIPLE // nw

    mesh = plsc.VectorSubcoreMesh(core_axis_name="c", subcore_axis_name="s")

    @functools.partial(
        pl.kernel,
        mesh=mesh,
        out_type=jax.ShapeDtypeStruct((NGROUP_OUT * OUT_SLAB,), jnp.float32),
        compiler_params=pltpu.CompilerParams(needs_layout_passes=False),
        scratch_types=[
            pltpu.VMEM((TOPK * DUAL * BATCH,), jnp.int32),
            pltpu.VMEM((DUAL * IN_SLAB,), jnp.float32),
            pltpu.VMEM((DUAL * IN_SLAB,), jnp.float32),
            pltpu.VMEM((TOPK * DUAL * OUT_SLAB,), jnp.float32),
            pltpu.SemaphoreType.DMA,
            pltpu.SemaphoreType.DMA,
            pltpu.SemaphoreType.DMA,
        ],
    )
    def body(vin_hbm, g_hbm, out_hbm, g_v, buf_a, buf_b, buf_out,
             sem_a, sem_b, sem_out):
        wid = lax.axis_index("s") * nc + lax.axis_index("c")
        pltpu.sync_copy(g_hbm, g_v)
        bufs = (buf_a, buf_b)
        sems = (sem_a, sem_b)

        def start_in(t, buf, sem):
            l = t // (LENGTH * HEADS)
            rem = t % (LENGTH * HEADS)
            g0 = l * (DUAL * LENGTH * HEADS) + rem
            for dd in range(DUAL):
                pltpu.async_copy(
                    vin_hbm.at[pl.ds((g0 + dd * (LENGTH * HEADS)) * IN_SLAB,
                                     IN_SLAB)],
                    buf.at[pl.ds(dd * IN_SLAB, IN_SLAB)], sem)

        def wait_in(buf, sem):
            for dd in range(DUAL):
                pltpu.make_async_copy(
                    vin_hbm.at[pl.ds(0, IN_SLAB)],
                    buf.at[pl.ds(dd * IN_SLAB, IN_SLAB)], sem).wait()

        def wait_out():
            for c in range(TOPK * DUAL):
                pltpu.make_async_copy(
                    buf_out.at[pl.ds(c * OUT_SLAB, OUT_SLAB)],
                    out_hbm.at[pl.ds(0, OUT_SLAB)], sem_out).wait()

        start_in(wid * tpw, buf_a, sem_a)

        @pl.loop(0, tpw, step=2)
        def outer(j):
            for bsel in range(2):
                je = j + bsel
                t = wid * tpw + je
                l = t // (LENGTH * HEADS)
                rem = t % (LENGTH * HEADS)
                s = rem // HEADS
                h = rem % HEADS
                buf_in = bufs[bsel]

                @pl.when(je + 1 < tpw)
                def _():
                    start_in(t + 1, bufs[1 - bsel], sems[1 - bsel])

                @pl.when(je > 0)
                def _():
                    wait_out()
                wait_in(buf_in, sems[bsel])

                def combo_body(c, carry2):
                    dprime = c // TOPK
                    k = c % TOPK

                    @plsc.parallel_loop(0, HDIM * (BATCH // 16), unroll=16)
                    def blk(v):
                        e = v // (BATCH // 16)
                        i = v % (BATCH // 16)
                        base = (e // 8) * (8 * POOL) + (e % 8) * BATCH
                        av = g_v[pl.ds(c * BATCH + i * 16, 16)] + base
                        val = plsc.load_gather(buf_in, [av])
                        buf_out[pl.ds(c * OUT_SLAB + e * BATCH + i * 16, 16)] = val
                    q = (l * DUAL + dprime) * (TOPK * LENGTH * HEADS) \
                        + (k * LENGTH + s) * HEADS + h
                    pltpu.async_copy(
                        buf_out.at[pl.ds(c * OUT_SLAB, OUT_SLAB)],
                        out_hbm.at[pl.ds(q * OUT_SLAB, OUT_SLAB)], sem_out)
                    return carry2

                lax.fori_loop(0, TOPK * DUAL, combo_body, 0)

        wait_out()

    return body(vin, g_addr)


def kernel(x_embed, prompt_idx, prompt):
    del x_embed
    idx32 = prompt_idx.astype(jnp.int32)

    vin = prompt.transpose(0, 1, 3, 4, 5, 2)
    vin = vin.reshape(NUM_LAYERS, DUAL, LENGTH, HEADS, 8, 8, 2, 128)
    vin = vin.transpose(0, 1, 2, 3, 4, 6, 5, 7)
    vin = vin.reshape(NGROUP_IN * IN_SLAB)

    bprime = jnp.arange(BATCH, dtype=jnp.int32)
    dprime = jnp.array([0, 0, 1, 1], dtype=jnp.int32)
    kk = jnp.array([0, 1, 0, 1], dtype=jnp.int32)
    m = 2 * bprime[None, :] + dprime[:, None]
    d = m // BATCH
    b = m % BATCH
    p = idx32[b, kk[:, None]]
    g_addr = (d * IN_SLAB + (p // 128) * (8 * 128) + p % 128).reshape(-1)

    vout = _sc_lane_gather(vin, g_addr)

    out = vout.reshape(NUM_LAYERS, DUAL, TOPK * LENGTH, HEADS, HDIM, BATCH)
    out = out.transpose(0, 5, 1, 2, 3, 4)
    return (prompt_idx, out)


# speedup vs baseline: 1.3767x; 1.0577x over previous
import functools

import jax
import jax.numpy as jnp
from jax import lax
from jax.experimental import pallas as pl
from jax.experimental.pallas import tpu as pltpu
from jax.experimental.pallas import tpu_sc as plsc

NUM_LAYERS = 12
DUAL = 2
POOL = 256
LENGTH = 8
HEADS = 12
HDIM = 64
BATCH = 128
TOPK = 2

NGROUP_IN = NUM_LAYERS * DUAL * LENGTH * HEADS
NGROUP_OUT = NUM_LAYERS * DUAL * TOPK * LENGTH * HEADS
IN_SLAB = HDIM * POOL
OUT_SLAB = HDIM * BATCH
NTRIPLE = NUM_LAYERS * LENGTH * HEADS


def _sc_lane_gather(vin, g_addr):
    info = plsc.get_sparse_core_info()
    nc, ns, nl = info.num_cores, info.num_subcores, info.num_lanes
    nw = nc * ns
    tpw = NTRIPLE // nw

    mesh = plsc.VectorSubcoreMesh(core_axis_name="c", subcore_axis_name="s")

    @functools.partial(
        pl.kernel,
        mesh=mesh,
        out_type=jax.ShapeDtypeStruct((NGROUP_OUT * OUT_SLAB,), jnp.float32),
        compiler_params=pltpu.CompilerParams(needs_layout_passes=False),
        scratch_types=[
            pltpu.VMEM((TOPK * DUAL * BATCH,), jnp.int32),
            pltpu.VMEM((DUAL * IN_SLAB,), jnp.float32),
            pltpu.VMEM((DUAL * IN_SLAB,), jnp.float32),
            pltpu.VMEM((TOPK * DUAL * OUT_SLAB,), jnp.float32),
            pltpu.SemaphoreType.DMA,
            pltpu.SemaphoreType.DMA,
            pltpu.SemaphoreType.DMA,
        ],
    )
    def body(vin_hbm, g_hbm, out_hbm, g_v, buf_a, buf_b, buf_out,
             sem_a, sem_b, sem_out):
        wid = lax.axis_index("s") * nc + lax.axis_index("c")
        pltpu.sync_copy(g_hbm, g_v)
        bufs = (buf_a, buf_b)
        sems = (sem_a, sem_b)

        def start_in(t, buf, sem):
            l = t // (LENGTH * HEADS)
            rem = t % (LENGTH * HEADS)
            g0 = l * (DUAL * LENGTH * HEADS) + rem
            for dd in range(DUAL):
                pltpu.async_copy(
                    vin_hbm.at[pl.ds((g0 + dd * (LENGTH * HEADS)) * IN_SLAB,
                                     IN_SLAB)],
                    buf.at[pl.ds(dd * IN_SLAB, IN_SLAB)], sem)

        def wait_in(buf, sem):
            for dd in range(DUAL):
                pltpu.make_async_copy(
                    vin_hbm.at[pl.ds(0, IN_SLAB)],
                    buf.at[pl.ds(dd * IN_SLAB, IN_SLAB)], sem).wait()

        def wait_out():
            for c in range(TOPK * DUAL):
                pltpu.make_async_copy(
                    buf_out.at[pl.ds(c * OUT_SLAB, OUT_SLAB)],
                    out_hbm.at[pl.ds(0, OUT_SLAB)], sem_out).wait()

        start_in(wid * tpw, buf_a, sem_a)

        @pl.loop(0, tpw, step=2)
        def outer(j):
            for bsel in range(2):
                je = j + bsel
                t = wid * tpw + je
                l = t // (LENGTH * HEADS)
                rem = t % (LENGTH * HEADS)
                s = rem // HEADS
                h = rem % HEADS
                buf_in = bufs[bsel]

                @pl.when(je + 1 < tpw)
                def _():
                    start_in(t + 1, bufs[1 - bsel], sems[1 - bsel])

                @pl.when(je > 0)
                def _():
                    wait_out()
                wait_in(buf_in, sems[bsel])

                def combo_body(c, carry2):
                    dprime = c // TOPK
                    k = c % TOPK

                    @plsc.parallel_loop(0, HDIM * (BATCH // 16), unroll=32)
                    def blk(v):
                        e = v // (BATCH // 16)
                        i = v % (BATCH // 16)
                        base = (e // 8) * (8 * POOL) + (e % 8) * BATCH
                        av = g_v[pl.ds(c * BATCH + i * 16, 16)] + base
                        val = plsc.load_gather(buf_in, [av])
                        buf_out[pl.ds(c * OUT_SLAB + e * BATCH + i * 16, 16)] = val
                    q = (l * DUAL + dprime) * (TOPK * LENGTH * HEADS) \
                        + (k * LENGTH + s) * HEADS + h
                    pltpu.async_copy(
                        buf_out.at[pl.ds(c * OUT_SLAB, OUT_SLAB)],
                        out_hbm.at[pl.ds(q * OUT_SLAB, OUT_SLAB)], sem_out)
                    return carry2

                lax.fori_loop(0, TOPK * DUAL, combo_body, 0)

        wait_out()

    return body(vin, g_addr)


def kernel(x_embed, prompt_idx, prompt):
    del x_embed
    idx32 = prompt_idx.astype(jnp.int32)

    vin = prompt.transpose(0, 1, 3, 4, 5, 2)
    vin = vin.reshape(NUM_LAYERS, DUAL, LENGTH, HEADS, 8, 8, 2, 128)
    vin = vin.transpose(0, 1, 2, 3, 4, 6, 5, 7)
    vin = vin.reshape(NGROUP_IN * IN_SLAB)

    bprime = jnp.arange(BATCH, dtype=jnp.int32)
    dprime = jnp.array([0, 0, 1, 1], dtype=jnp.int32)
    kk = jnp.array([0, 1, 0, 1], dtype=jnp.int32)
    m = 2 * bprime[None, :] + dprime[:, None]
    d = m // BATCH
    b = m % BATCH
    p = idx32[b, kk[:, None]]
    g_addr = (d * IN_SLAB + (p // 128) * (8 * 128) + p % 128).reshape(-1)

    vout = _sc_lane_gather(vin, g_addr)

    out = vout.reshape(NUM_LAYERS, DUAL, TOPK * LENGTH, HEADS, HDIM, BATCH)
    out = out.transpose(0, 5, 1, 2, 3, 4)
    return (prompt_idx, out)


# speedup vs baseline: 1.3890x; 1.0090x over previous
import functools

import jax
import jax.numpy as jnp
from jax import lax
from jax.experimental import pallas as pl
from jax.experimental.pallas import tpu as pltpu
from jax.experimental.pallas import tpu_sc as plsc

NUM_LAYERS = 12
DUAL = 2
POOL = 256
LENGTH = 8
HEADS = 12
HDIM = 64
BATCH = 128
TOPK = 2

NGROUP_IN = NUM_LAYERS * DUAL * LENGTH * HEADS
NGROUP_OUT = NUM_LAYERS * DUAL * TOPK * LENGTH * HEADS
IN_SLAB = HDIM * POOL
OUT_SLAB = HDIM * BATCH
NTRIPLE = NUM_LAYERS * LENGTH * HEADS


def _sc_lane_gather(vin, g_addr):
    info = plsc.get_sparse_core_info()
    nc, ns, nl = info.num_cores, info.num_subcores, info.num_lanes
    nw = nc * ns
    tpw = NTRIPLE // nw

    mesh = plsc.VectorSubcoreMesh(core_axis_name="c", subcore_axis_name="s")

    @functools.partial(
        pl.kernel,
        mesh=mesh,
        out_type=jax.ShapeDtypeStruct((NGROUP_OUT * OUT_SLAB,), jnp.float32),
        compiler_params=pltpu.CompilerParams(needs_layout_passes=False),
        scratch_types=[
            pltpu.VMEM((TOPK * DUAL * BATCH,), jnp.int32),
            pltpu.VMEM((DUAL * IN_SLAB,), jnp.float32),
            pltpu.VMEM((DUAL * IN_SLAB,), jnp.float32),
            pltpu.VMEM((TOPK * DUAL * OUT_SLAB,), jnp.float32),
            pltpu.SemaphoreType.DMA,
            pltpu.SemaphoreType.DMA,
            pltpu.SemaphoreType.DMA,
        ],
    )
    def body(vin_hbm, g_hbm, out_hbm, g_v, buf_a, buf_b, buf_out,
             sem_a, sem_b, sem_out):
        wid = lax.axis_index("s") * nc + lax.axis_index("c")
        pltpu.sync_copy(g_hbm, g_v)
        bufs = (buf_a, buf_b)
        sems = (sem_a, sem_b)

        def start_in(t, buf, sem):
            l = t // (LENGTH * HEADS)
            rem = t % (LENGTH * HEADS)
            g0 = l * (DUAL * LENGTH * HEADS) + rem
            for dd in range(DUAL):
                pltpu.async_copy(
                    vin_hbm.at[pl.ds((g0 + dd * (LENGTH * HEADS)) * IN_SLAB,
                                     IN_SLAB)],
                    buf.at[pl.ds(dd * IN_SLAB, IN_SLAB)], sem)

        def wait_in(buf, sem):
            for dd in range(DUAL):
                pltpu.make_async_copy(
                    vin_hbm.at[pl.ds(0, IN_SLAB)],
                    buf.at[pl.ds(dd * IN_SLAB, IN_SLAB)], sem).wait()

        def wait_out():
            for c in range(TOPK * DUAL):
                pltpu.make_async_copy(
                    buf_out.at[pl.ds(c * OUT_SLAB, OUT_SLAB)],
                    out_hbm.at[pl.ds(0, OUT_SLAB)], sem_out).wait()

        start_in(wid * tpw, buf_a, sem_a)

        @pl.loop(0, tpw, step=2)
        def outer(j):
            for bsel in range(2):
                je = j + bsel
                t = wid * tpw + je
                l = t // (LENGTH * HEADS)
                rem = t % (LENGTH * HEADS)
                s = rem // HEADS
                h = rem % HEADS
                buf_in = bufs[bsel]

                @pl.when(je + 1 < tpw)
                def _():
                    start_in(t + 1, bufs[1 - bsel], sems[1 - bsel])

                @pl.when(je > 0)
                def _():
                    wait_out()
                wait_in(buf_in, sems[bsel])

                def combo_body(c, carry2):
                    dprime = c // TOPK
                    k = c % TOPK

                    @plsc.parallel_loop(0, HDIM * (BATCH // 16), unroll=64)
                    def blk(v):
                        e = v // (BATCH // 16)
                        i = v % (BATCH // 16)
                        base = (e // 8) * (8 * POOL) + (e % 8) * BATCH
                        av = g_v[pl.ds(c * BATCH + i * 16, 16)] + base
                        val = plsc.load_gather(buf_in, [av])
                        buf_out[pl.ds(c * OUT_SLAB + e * BATCH + i * 16, 16)] = val
                    q = (l * DUAL + dprime) * (TOPK * LENGTH * HEADS) \
                        + (k * LENGTH + s) * HEADS + h
                    pltpu.async_copy(
                        buf_out.at[pl.ds(c * OUT_SLAB, OUT_SLAB)],
                        out_hbm.at[pl.ds(q * OUT_SLAB, OUT_SLAB)], sem_out)
                    return carry2

                lax.fori_loop(0, TOPK * DUAL, combo_body, 0)

        wait_out()

    return body(vin, g_addr)


def kernel(x_embed, prompt_idx, prompt):
    del x_embed
    idx32 = prompt_idx.astype(jnp.int32)

    vin = prompt.transpose(0, 1, 3, 4, 5, 2)
    vin = vin.reshape(NUM_LAYERS, DUAL, LENGTH, HEADS, 8, 8, 2, 128)
    vin = vin.transpose(0, 1, 2, 3, 4, 6, 5, 7)
    vin = vin.reshape(NGROUP_IN * IN_SLAB)

    bprime = jnp.arange(BATCH, dtype=jnp.int32)
    dprime = jnp.array([0, 0, 1, 1], dtype=jnp.int32)
    kk = jnp.array([0, 1, 0, 1], dtype=jnp.int32)
    m = 2 * bprime[None, :] + dprime[:, None]
    d = m // BATCH
    b = m % BATCH
    p = idx32[b, kk[:, None]]
    g_addr = (d * IN_SLAB + (p // 128) * (8 * 128) + p % 128).reshape(-1)

    vout = _sc_lane_gather(vin, g_addr)

    out = vout.reshape(NUM_LAYERS, DUAL, TOPK * LENGTH, HEADS, HDIM, BATCH)
    out = out.transpose(0, 5, 1, 2, 3, 4)
    return (prompt_idx, out)
